# Initial kernel scaffold; baseline (speedup 1.0000x reference)
import jax
import jax.numpy as jnp
from jax.experimental import pallas as pl


def kernel(ufea, vfea, uv_edge_index, uv_values, vu_edge_index, vu_values, Wu0, bu0, Wi0, bi0, Wu1, bu1, Wi1, bi1):
    raise NotImplementedError("write your pallas kernel here")


# speedup vs baseline: 2.8608x; 2.8608x over previous
import functools

import jax
import jax.numpy as jnp
from jax import lax
from jax.experimental import pallas as pl
from jax.experimental.pallas import tpu as pltpu
from jax.experimental.pallas import tpu_sc as plsc

N = 10000
D = 128
E = 320000
NC = 2
NS = 16
NW = NC * NS
CH = 128
CHUNKS = -(-E // (NW * CH))
E_PAD = NW * CH * CHUNKS
NP = 10240
ROWS_PER_TILE = NP // NS
ZR = 128
LANES = 16


def _spmm_sc_kernel(rows_hbm, cols_hbm, vals_hbm, x_hbm, out_hbm,
                    colv, rowv, valv, gbuf, zbuf, acc):
    c = lax.axis_index("c")
    s = lax.axis_index("s")
    w = s * NC + c

    zero = jnp.zeros((LANES,), jnp.float32)

    def zb(i, carry):
        for j in range(D // LANES):
            zbuf[i, pl.ds(j * LANES, LANES)] = zero
        return carry

    lax.fori_loop(0, ZR, zb, 0)
    base_rows = s * ROWS_PER_TILE
    for k in range(ROWS_PER_TILE // ZR):
        pltpu.sync_copy(zbuf, acc.at[pl.ds(base_rows + k * ZR, ZR)])
    plsc.subcore_barrier()

    edge_base = w * (CHUNKS * CH)

    def body(t, carry):
        base = edge_base + t * CH
        pltpu.sync_copy(rows_hbm.at[pl.ds(base, CH)], rowv)
        pltpu.sync_copy(cols_hbm.at[pl.ds(base, CH)], colv)
        pltpu.sync_copy(vals_hbm.at[pl.ds(base, CH)], valv)
        pltpu.sync_copy(x_hbm.at[colv], gbuf)

        def mul(g, inner):
            vv = valv[pl.ds(g * LANES, LANES)]
            for l in range(LANES):
                v = vv[l]
                i = g * LANES + l
                for j in range(D // LANES):
                    sl = (i, pl.ds(j * LANES, LANES))
                    gbuf[sl] = gbuf[sl] * v
            return inner

        lax.fori_loop(0, CH // LANES, mul, 0)
        pltpu.sync_copy(gbuf, acc.at[rowv], add=True)
        return carry

    lax.fori_loop(0, CHUNKS, body, 0)
    plsc.subcore_barrier()

    for k in range(ROWS_PER_TILE // ZR):
        off = c * NP + base_rows + k * ZR
        pltpu.sync_copy(acc.at[pl.ds(base_rows + k * ZR, ZR)],
                        out_hbm.at[pl.ds(off, ZR)])


_spmm_sc = functools.partial(
    pl.kernel,
    out_type=jax.ShapeDtypeStruct((NC * NP, D), jnp.float32),
    mesh=plsc.VectorSubcoreMesh(core_axis_name="c", subcore_axis_name="s"),
    scratch_types=[
        pltpu.VMEM((CH,), jnp.int32),
        pltpu.VMEM((CH,), jnp.int32),
        pltpu.VMEM((CH,), jnp.float32),
        pltpu.VMEM((CH, D), jnp.float32),
        pltpu.VMEM((ZR, D), jnp.float32),
        pltpu.VMEM_SHARED((NP, D), jnp.float32),
    ],
)(_spmm_sc_kernel)


def _spmm(rows, cols, vals, x):
    return _spmm_sc(rows, cols, vals, x)


BLK = 80
GRID = N // BLK
POFF = NP // BLK


def _combine_body(p0_ref, p1_ref, o_ref):
    o_ref[...] = p0_ref[...] + p1_ref[...]


_combine_call = pl.pallas_call(
    _combine_body,
    grid=(GRID,),
    in_specs=[
        pl.BlockSpec((BLK, D), lambda i: (i, 0)),
        pl.BlockSpec((BLK, D), lambda i: (i + POFF, 0)),
    ],
    out_specs=pl.BlockSpec((BLK, D), lambda i: (i, 0)),
    out_shape=jax.ShapeDtypeStruct((N, D), jnp.float32),
)


def _combine(p):
    return _combine_call(p, p)


def _linear_body(p0_ref, p1_ref, u_ref, wt_ref, b_ref, o_ref):
    ho = p0_ref[...] + p1_ref[...]
    acc = jnp.dot(ho, wt_ref[:D, :], preferred_element_type=jnp.float32)
    acc += jnp.dot(u_ref[...], wt_ref[D:, :], preferred_element_type=jnp.float32)
    o_ref[...] = jnp.maximum(acc + b_ref[...], 0.0)


_linear_call = pl.pallas_call(
    _linear_body,
    grid=(GRID,),
    in_specs=[
        pl.BlockSpec((BLK, D), lambda i: (i, 0)),
        pl.BlockSpec((BLK, D), lambda i: (i + POFF, 0)),
        pl.BlockSpec((BLK, D), lambda i: (i, 0)),
        pl.BlockSpec((2 * D, D), lambda i: (0, 0)),
        pl.BlockSpec((1, D), lambda i: (0, 0)),
    ],
    out_specs=pl.BlockSpec((BLK, D), lambda i: (i, 0)),
    out_shape=jax.ShapeDtypeStruct((N, D), jnp.float32),
)


def _linear(ho_partials, learn, W, b):
    wt = W.T
    b2 = b.reshape(1, D)
    return _linear_call(ho_partials, ho_partials, learn, wt, b2)


def _pad_edges(edge_index, values):
    rows = edge_index[0]
    cols = edge_index[1]
    pad = E_PAD - E
    rows = jnp.pad(rows, (0, pad))
    cols = jnp.pad(cols, (0, pad))
    vals = jnp.pad(values, (0, pad))
    return rows, cols, vals


def kernel(ufea, vfea, uv_edge_index, uv_values, vu_edge_index, vu_values,
           Wu0, bu0, Wi0, bi0, Wu1, bu1, Wi1, bi1):
    uv_r, uv_c, uv_v = _pad_edges(uv_edge_index, uv_values)
    vu_r, vu_c, vu_v = _pad_edges(vu_edge_index, vu_values)

    learn_user = ufea
    learn_item = vfea
    for (Wu, bu, Wi, bi) in ((Wu0, bu0, Wi0, bi0), (Wu1, bu1, Wi1, bi1)):
        t_u = _combine(_spmm(vu_r, vu_c, vu_v, learn_user))
        user_ho_p = _spmm(uv_r, uv_c, uv_v, t_u)
        t_i = _combine(_spmm(uv_r, uv_c, uv_v, learn_item))
        item_ho_p = _spmm(vu_r, vu_c, vu_v, t_i)
        learn_user = _linear(user_ho_p, learn_user, Wu, bu)
        learn_item = _linear(item_ho_p, learn_item, Wi, bi)
    return (learn_user, learn_item)
